# Initial kernel scaffold; baseline (speedup 1.0000x reference)
import jax
import jax.numpy as jnp
from jax.experimental import pallas as pl

EMB = 8
LANES = 128
VECS_PER_ROW = LANES // EMB


def _vq_body(x_ref, bd_ref, bw_ref, cvec_ref, cbrow_ref, q_ref, cb_ref):
    x = x_ref[...]
    dist = jax.lax.dot(x, bd_ref[...], preferred_element_type=jnp.float32)
    dist = dist + cvec_ref[...]
    lane = jax.lax.broadcasted_iota(jnp.int32, dist.shape, 1)
    g = dist
    for k in (1, 2, 4):
        plus = jnp.roll(g, k, axis=1)
        minus = jnp.roll(g, -k, axis=1)
        partner = jnp.where((lane & k) != 0, plus, minus)
        g = jnp.minimum(g, partner)
    onehot = (dist == g).astype(jnp.float32)
    q_ref[...] = jax.lax.dot(onehot, bw_ref[...],
                             preferred_element_type=jnp.float32)
    cb_ref[...] = jnp.broadcast_to(cbrow_ref[...], cb_ref.shape)


def kernel(latent, W):
    B, LSZ, D = latent.shape
    N = (B * LSZ * D) // LANES
    x2 = latent.reshape(N, LANES)

    eye = jnp.eye(VECS_PER_ROW, dtype=jnp.float32)
    bd = jnp.kron(eye, (-2.0) * W.T.astype(jnp.float32))
    bw = jnp.kron(eye, W.astype(jnp.float32))
    eps = jnp.float32(2.0 ** -18)
    bias = jnp.arange(EMB, dtype=jnp.float32) * eps
    cvec = jnp.tile(jnp.sum(W * W, axis=1) + bias, VECS_PER_ROW)
    cvec = cvec.reshape(1, LANES)
    cbrow = jnp.tile(W.reshape(-1), 2).reshape(1, LANES)

    R = 512
    grid = (N // R,)
    q2, cb2 = pl.pallas_call(
        _vq_body,
        grid=grid,
        in_specs=[
            pl.BlockSpec((R, LANES), lambda i: (i, 0)),
            pl.BlockSpec((LANES, LANES), lambda i: (0, 0)),
            pl.BlockSpec((LANES, LANES), lambda i: (0, 0)),
            pl.BlockSpec((1, LANES), lambda i: (0, 0)),
            pl.BlockSpec((1, LANES), lambda i: (0, 0)),
        ],
        out_specs=[
            pl.BlockSpec((R, LANES), lambda i: (i, 0)),
            pl.BlockSpec((2 * R, LANES), lambda i: (i, 0)),
        ],
        out_shape=[
            jax.ShapeDtypeStruct((N, LANES), jnp.float32),
            jax.ShapeDtypeStruct((2 * N, LANES), jnp.float32),
        ],
    )(x2, bd, bw, cvec, cbrow)

    q = q2.reshape(B, LSZ, D)
    cb = cb2.reshape(B, EMB, D)
    return (q, q, cb)


# speedup vs baseline: 2.2398x; 2.2398x over previous
import jax
import jax.numpy as jnp
from jax.experimental import pallas as pl

EMB = 8
LANES = 128
VECS_PER_ROW = LANES // EMB


def _vq_body(x_ref, bd_ref, bw_ref, cvec_ref, cbrow_ref, q_ref, cb_ref):
    x = x_ref[...]
    dist = jax.lax.dot(x, bd_ref[...], preferred_element_type=jnp.float32)
    dist = dist + cvec_ref[...]
    lane = jax.lax.broadcasted_iota(jnp.int32, dist.shape, 1)
    g = dist
    for k in (1, 2, 4):
        plus = jnp.roll(g, k, axis=1)
        minus = jnp.roll(g, -k, axis=1)
        partner = jnp.where((lane & k) != 0, plus, minus)
        g = jnp.minimum(g, partner)
    onehot = (dist == g).astype(jnp.float32)
    q_ref[...] = jax.lax.dot(onehot, bw_ref[...],
                             preferred_element_type=jnp.float32)
    cb_ref[...] = jnp.broadcast_to(cbrow_ref[...], cb_ref.shape)


def kernel(latent, W):
    B, LSZ, D = latent.shape
    N = (B * LSZ * D) // LANES
    x2 = latent.reshape(N, LANES)

    eye = jnp.eye(VECS_PER_ROW, dtype=jnp.float32)
    bd = jnp.kron(eye, (-2.0) * W.T.astype(jnp.float32))
    bw = jnp.kron(eye, W.astype(jnp.float32))
    cvec = jnp.tile(jnp.sum(W * W, axis=1), VECS_PER_ROW).reshape(1, LANES)
    cbrow = jnp.tile(W.reshape(-1), 2).reshape(1, LANES)

    R = 512
    grid = (N // R,)
    q2, cb2 = pl.pallas_call(
        _vq_body,
        grid=grid,
        in_specs=[
            pl.BlockSpec((R, LANES), lambda i: (i, 0)),
            pl.BlockSpec((LANES, LANES), lambda i: (0, 0)),
            pl.BlockSpec((LANES, LANES), lambda i: (0, 0)),
            pl.BlockSpec((1, LANES), lambda i: (0, 0)),
            pl.BlockSpec((1, LANES), lambda i: (0, 0)),
        ],
        out_specs=[
            pl.BlockSpec((R, LANES), lambda i: (i, 0)),
            pl.BlockSpec((2 * R, LANES), lambda i: (i, 0)),
        ],
        out_shape=[
            jax.ShapeDtypeStruct((N, LANES), jnp.float32),
            jax.ShapeDtypeStruct((2 * N, LANES), jnp.float32),
        ],
    )(x2, bd, bw, cvec, cbrow)

    q = q2.reshape(B, LSZ, D)
    cb = cb2.reshape(B, EMB, D)
    return (q, q, cb)


# speedup vs baseline: 32.7589x; 14.6255x over previous
import jax
import jax.numpy as jnp
from jax.experimental import pallas as pl

EMB = 8
LSZ = 4


def _vq_body(x_ref, wm2_ref, wt_ref, wn_ref, q_ref, p_ref, cb_ref):
    wm2 = wm2_ref[...]
    wt = wt_ref[...]
    wn = wn_ref[...]
    for l in range(LSZ):
        x = x_ref[l]
        dots = jax.lax.dot(wm2, x, preferred_element_type=jnp.float32)
        dist = dots + wn
        best = dist[0:1, :]
        bidx = jnp.zeros_like(best, dtype=jnp.int32)
        for e in range(1, EMB):
            row = dist[e:e + 1, :]
            lt = row < best
            best = jnp.where(lt, row, best)
            bidx = jnp.where(lt, e, bidx)
        rowiota = jax.lax.broadcasted_iota(jnp.int32, dist.shape, 0)
        onehot = (rowiota == bidx).astype(jnp.float32)
        q = jax.lax.dot(wt, onehot, preferred_element_type=jnp.float32)
        q_ref[l] = q
        p_ref[l] = q
    for e in range(EMB):
        cb_ref[e] = jnp.broadcast_to(wt[:, e:e + 1], cb_ref.shape[1:])


def kernel(latent, W):
    B = latent.shape[0]
    xt = latent.transpose(1, 2, 0)
    wm2 = (-2.0) * W
    wt = W.T
    wn = jnp.sum(W * W, axis=1, keepdims=True)

    C = 2048
    grid = (B // C,)
    qt, pt, cbt = pl.pallas_call(
        _vq_body,
        grid=grid,
        in_specs=[
            pl.BlockSpec((LSZ, EMB, C), lambda i: (0, 0, i)),
            pl.BlockSpec((EMB, EMB), lambda i: (0, 0)),
            pl.BlockSpec((EMB, EMB), lambda i: (0, 0)),
            pl.BlockSpec((EMB, 1), lambda i: (0, 0)),
        ],
        out_specs=[
            pl.BlockSpec((LSZ, EMB, C), lambda i: (0, 0, i)),
            pl.BlockSpec((LSZ, EMB, C), lambda i: (0, 0, i)),
            pl.BlockSpec((EMB, EMB, C), lambda i: (0, 0, i)),
        ],
        out_shape=[
            jax.ShapeDtypeStruct((LSZ, EMB, B), jnp.float32),
            jax.ShapeDtypeStruct((LSZ, EMB, B), jnp.float32),
            jax.ShapeDtypeStruct((EMB, EMB, B), jnp.float32),
        ],
    )(xt, wm2, wt, wn)

    q = qt.transpose(2, 0, 1)
    p = pt.transpose(2, 0, 1)
    cb = cbt.transpose(2, 0, 1)
    return (p, q, cb)


# speedup vs baseline: 62.2894x; 1.9015x over previous
import jax
import jax.numpy as jnp
from jax.experimental import pallas as pl

EMB = 8
LSZ = 4


def _vq_body(x_ref, wm2_ref, wt_ref, wn_ref, q_ref, p_ref, cb_ref):
    wm2 = wm2_ref[...]
    wt = wt_ref[...]
    wn = wn_ref[...]
    for l in range(LSZ):
        x = x_ref[l]
        dots = jax.lax.dot(wm2, x, preferred_element_type=jnp.float32)
        dist = dots + wn
        g = dist
        for k in (1, 2, 4):
            g = jnp.minimum(g, jnp.roll(g, k, axis=0))
        onehot = (dist == g).astype(jnp.float32)
        q = jax.lax.dot(wt, onehot, preferred_element_type=jnp.float32)
        q_ref[l] = q
        p_ref[l] = q
    for e in range(EMB):
        cb_ref[e] = jnp.broadcast_to(wt[:, e:e + 1], cb_ref.shape[1:])


def kernel(latent, W):
    B = latent.shape[0]
    xt = latent.transpose(1, 2, 0)
    wm2 = (-2.0) * W
    wt = W.T
    wn = jnp.sum(W * W, axis=1, keepdims=True)

    C = 8192
    grid = (B // C,)
    qt, pt, cbt = pl.pallas_call(
        _vq_body,
        grid=grid,
        in_specs=[
            pl.BlockSpec((LSZ, EMB, C), lambda i: (0, 0, i)),
            pl.BlockSpec((EMB, EMB), lambda i: (0, 0)),
            pl.BlockSpec((EMB, EMB), lambda i: (0, 0)),
            pl.BlockSpec((EMB, 1), lambda i: (0, 0)),
        ],
        out_specs=[
            pl.BlockSpec((LSZ, EMB, C), lambda i: (0, 0, i)),
            pl.BlockSpec((LSZ, EMB, C), lambda i: (0, 0, i)),
            pl.BlockSpec((EMB, EMB, C), lambda i: (0, 0, i)),
        ],
        out_shape=[
            jax.ShapeDtypeStruct((LSZ, EMB, B), jnp.float32),
            jax.ShapeDtypeStruct((LSZ, EMB, B), jnp.float32),
            jax.ShapeDtypeStruct((EMB, EMB, B), jnp.float32),
        ],
    )(xt, wm2, wt, wn)

    q = qt.transpose(2, 0, 1)
    p = pt.transpose(2, 0, 1)
    cb = cbt.transpose(2, 0, 1)
    return (p, q, cb)


# speedup vs baseline: 66.9508x; 1.0748x over previous
import jax
import jax.numpy as jnp
from jax.experimental import pallas as pl

EMB = 8
LSZ = 4


def _vq_body(x_ref, wm2_ref, wt_ref, wn_ref, q_ref, p_ref, cb_ref):
    wm2 = wm2_ref[...]
    wt = wt_ref[...]
    wn = wn_ref[...]
    for l in range(LSZ):
        x = x_ref[l]
        dots = jax.lax.dot(wm2, x, preferred_element_type=jnp.float32)
        dist = dots + wn
        g = dist
        for k in (1, 2, 4):
            g = jnp.minimum(g, jnp.roll(g, k, axis=0))
        onehot = (dist == g).astype(jnp.float32)
        q = jax.lax.dot(wt, onehot, preferred_element_type=jnp.float32)
        q_ref[l] = q
        p_ref[l] = q
    for e in range(EMB):
        cb_ref[e] = jnp.broadcast_to(wt[:, e:e + 1], cb_ref.shape[1:])


def kernel(latent, W):
    B = latent.shape[0]
    xt = latent.transpose(1, 2, 0)
    wm2 = (-2.0) * W
    wt = W.T
    wn = jnp.sum(W * W, axis=1, keepdims=True)

    C = 16384
    grid = (B // C,)
    qt, pt, cbt = pl.pallas_call(
        _vq_body,
        grid=grid,
        in_specs=[
            pl.BlockSpec((LSZ, EMB, C), lambda i: (0, 0, i)),
            pl.BlockSpec((EMB, EMB), lambda i: (0, 0)),
            pl.BlockSpec((EMB, EMB), lambda i: (0, 0)),
            pl.BlockSpec((EMB, 1), lambda i: (0, 0)),
        ],
        out_specs=[
            pl.BlockSpec((LSZ, EMB, C), lambda i: (0, 0, i)),
            pl.BlockSpec((LSZ, EMB, C), lambda i: (0, 0, i)),
            pl.BlockSpec((EMB, EMB, C), lambda i: (0, 0, i)),
        ],
        out_shape=[
            jax.ShapeDtypeStruct((LSZ, EMB, B), jnp.float32),
            jax.ShapeDtypeStruct((LSZ, EMB, B), jnp.float32),
            jax.ShapeDtypeStruct((EMB, EMB, B), jnp.float32),
        ],
    )(xt, wm2, wt, wn)

    q = qt.transpose(2, 0, 1)
    p = pt.transpose(2, 0, 1)
    cb = cbt.transpose(2, 0, 1)
    return (p, q, cb)
